# initial kernel scaffold (unmeasured)
import jax
import jax.numpy as jnp
from jax import lax
from jax.experimental import pallas as pl
from jax.experimental.pallas import tpu as pltpu

N_DEV = 4


def kernel(x, dest):
    m, n = x.shape
    xb = x.astype(jnp.bfloat16)
    d2 = dest.reshape(8, m // 8).astype(jnp.int32)

    def body(x_ref, d_ref, xg_ref, dg_ref, comm_x, comm_d, sx, rx, sd, rd, cp_sem):
        my_x = lax.axis_index("x")
        my_y = lax.axis_index("y")
        my_z = lax.axis_index("z")
        left = (my_y - 1) % N_DEV
        right = (my_y + 1) % N_DEV

        comm_x[0] = x_ref[...]
        comm_d[0] = d_ref[...]
        cp = pltpu.make_async_copy(x_ref, xg_ref.at[0], cp_sem)
        cp.start()
        cp.wait()
        cp = pltpu.make_async_copy(d_ref, dg_ref.at[0], cp_sem)
        cp.start()
        cp.wait()

        barrier = pltpu.get_barrier_semaphore()
        for nbr in (left, right):
            pl.semaphore_signal(
                barrier,
                inc=1,
                device_id=(my_x, nbr, my_z),
                device_id_type=pl.DeviceIdType.MESH,
            )
        pl.semaphore_wait(barrier, 2)

        for h in range(N_DEV - 1):
            s, r = h % 2, (h + 1) % 2
            rdx = pltpu.make_async_remote_copy(
                src_ref=comm_x.at[s],
                dst_ref=comm_x.at[r],
                send_sem=sx.at[s],
                recv_sem=rx.at[r],
                device_id=(my_x, right, my_z),
                device_id_type=pl.DeviceIdType.MESH,
            )
            rdd = pltpu.make_async_remote_copy(
                src_ref=comm_d.at[s],
                dst_ref=comm_d.at[r],
                send_sem=sd.at[s],
                recv_sem=rd.at[r],
                device_id=(my_x, right, my_z),
                device_id_type=pl.DeviceIdType.MESH,
            )
            rdx.start()
            rdd.start()
            rdx.wait()
            rdd.wait()
            cpx = pltpu.make_async_copy(comm_x.at[r], xg_ref.at[h + 1], cp_sem)
            cpx.start()
            cpx.wait()
            cpd = pltpu.make_async_copy(comm_d.at[r], dg_ref.at[h + 1], cp_sem)
            cpd.start()
            cpd.wait()

    xg, dg = pl.pallas_call(
        body,
        out_shape=[
            jax.ShapeDtypeStruct((N_DEV, m, n), jnp.bfloat16),
            jax.ShapeDtypeStruct((N_DEV, 8, m // 8), jnp.int32),
        ],
        in_specs=[
            pl.BlockSpec(memory_space=pltpu.VMEM),
            pl.BlockSpec(memory_space=pltpu.VMEM),
        ],
        out_specs=[
            pl.BlockSpec(memory_space=pltpu.ANY),
            pl.BlockSpec(memory_space=pltpu.ANY),
        ],
        scratch_shapes=[
            pltpu.VMEM((2, m, n), jnp.bfloat16),
            pltpu.VMEM((2, 8, m // 8), jnp.int32),
            pltpu.SemaphoreType.DMA((2,)),
            pltpu.SemaphoreType.DMA((2,)),
            pltpu.SemaphoreType.DMA((2,)),
            pltpu.SemaphoreType.DMA((2,)),
            pltpu.SemaphoreType.DMA,
        ],
        compiler_params=pltpu.CompilerParams(collective_id=0),
    )(xb, d2)

    my_y = lax.axis_index("y")
    slot_for_origin = (my_y - jnp.arange(N_DEV)) % N_DEV
    xg_o = jnp.take(xg, slot_for_origin, axis=0).reshape(N_DEV * m, n)
    dg_o = jnp.take(dg, slot_for_origin, axis=0).reshape(N_DEV * m)
    idx = jnp.nonzero(dg_o == my_y, size=m, fill_value=0)[0]
    return xg_o[idx].astype(jnp.float32)


# baseline (device time: 673305 ns/iter reference)
import jax
import jax.numpy as jnp
from jax import lax
from jax.experimental import pallas as pl
from jax.experimental.pallas import tpu as pltpu

N_DEV = 4


def kernel(x, dest):
    m, n = x.shape
    xb = x.astype(jnp.bfloat16)
    d2 = dest.reshape(8, m // 8).astype(jnp.int32)

    def body(x_ref, d_ref, xg_ref, dg_ref, comm_x, comm_d, sx, rx, sd, rd, cp_sem):
        my_x = lax.axis_index("x")
        my_y = lax.axis_index("y")
        my_z = lax.axis_index("z")
        left = (my_y - 1) % N_DEV
        right = (my_y + 1) % N_DEV

        comm_x[0] = x_ref[...]
        comm_d[0] = d_ref[...]
        cp = pltpu.make_async_copy(x_ref, xg_ref.at[0], cp_sem)
        cp.start()
        cp.wait()
        cp = pltpu.make_async_copy(d_ref, dg_ref.at[0], cp_sem)
        cp.start()
        cp.wait()

        barrier = pltpu.get_barrier_semaphore()
        for nbr in (left, right):
            pl.semaphore_signal(
                barrier,
                inc=1,
                device_id=(my_x, nbr, my_z),
                device_id_type=pl.DeviceIdType.MESH,
            )
        pl.semaphore_wait(barrier, 2)

        for h in range(N_DEV - 1):
            s, r = h % 2, (h + 1) % 2
            rdx = pltpu.make_async_remote_copy(
                src_ref=comm_x.at[s],
                dst_ref=comm_x.at[r],
                send_sem=sx.at[s],
                recv_sem=rx.at[r],
                device_id=(my_x, right, my_z),
                device_id_type=pl.DeviceIdType.MESH,
            )
            rdd = pltpu.make_async_remote_copy(
                src_ref=comm_d.at[s],
                dst_ref=comm_d.at[r],
                send_sem=sd.at[s],
                recv_sem=rd.at[r],
                device_id=(my_x, right, my_z),
                device_id_type=pl.DeviceIdType.MESH,
            )
            rdx.start()
            rdd.start()
            rdx.wait()
            rdd.wait()
            cpx = pltpu.make_async_copy(comm_x.at[r], xg_ref.at[h + 1], cp_sem)
            cpx.start()
            cpx.wait()
            cpd = pltpu.make_async_copy(comm_d.at[r], dg_ref.at[h + 1], cp_sem)
            cpd.start()
            cpd.wait()

    xg, dg = pl.pallas_call(
        body,
        out_shape=[
            jax.ShapeDtypeStruct((N_DEV, m, n), jnp.bfloat16),
            jax.ShapeDtypeStruct((N_DEV, 8, m // 8), jnp.int32),
        ],
        in_specs=[
            pl.BlockSpec(memory_space=pltpu.VMEM),
            pl.BlockSpec(memory_space=pltpu.VMEM),
        ],
        out_specs=[
            pl.BlockSpec(memory_space=pl.ANY),
            pl.BlockSpec(memory_space=pl.ANY),
        ],
        scratch_shapes=[
            pltpu.VMEM((2, m, n), jnp.bfloat16),
            pltpu.VMEM((2, 8, m // 8), jnp.int32),
            pltpu.SemaphoreType.DMA((2,)),
            pltpu.SemaphoreType.DMA((2,)),
            pltpu.SemaphoreType.DMA((2,)),
            pltpu.SemaphoreType.DMA((2,)),
            pltpu.SemaphoreType.DMA,
        ],
        compiler_params=pltpu.CompilerParams(collective_id=0),
    )(xb, d2)

    my_y = lax.axis_index("y")
    slot_for_origin = (my_y - jnp.arange(N_DEV)) % N_DEV
    xg_o = jnp.take(xg, slot_for_origin, axis=0).reshape(N_DEV * m, n)
    dg_o = jnp.take(dg, slot_for_origin, axis=0).reshape(N_DEV * m)
    idx = jnp.nonzero(dg_o == my_y, size=m, fill_value=0)[0]
    return xg_o[idx].astype(jnp.float32)


# device time: 209758 ns/iter; 3.2099x vs baseline; 3.2099x over previous
import jax
import jax.numpy as jnp
from jax import lax
from jax.experimental import pallas as pl
from jax.experimental.pallas import tpu as pltpu

N_DEV = 4
P = 1152


def kernel(x, dest):
    m, n = x.shape
    my_y = lax.axis_index("y")

    xb = jnp.concatenate(
        [x.astype(jnp.bfloat16), jnp.zeros((1, n), jnp.bfloat16)], axis=0
    )
    counts = jnp.bincount(dest, length=N_DEV).astype(jnp.int32)
    slot_dst = (my_y + jnp.arange(N_DEV)) % N_DEV
    idx = jax.vmap(
        lambda d: jnp.nonzero(dest == d, size=P, fill_value=m)[0]
    )(slot_dst)
    sendbuf = xb[idx.reshape(-1)].reshape(N_DEV, P, n)
    sendcnt = jnp.zeros((8, 128), jnp.int32).at[0, :N_DEV].set(counts)

    def body(sb_ref, sc_ref, recv_ref, rc_ref, ds, dr, cs, cr, cp_sem):
        my_x = lax.axis_index("x")
        yy = lax.axis_index("y")
        my_z = lax.axis_index("z")

        cp = pltpu.make_async_copy(sb_ref.at[0], recv_ref.at[0], cp_sem)
        cp.start()
        cp2 = pltpu.make_async_copy(sc_ref, rc_ref.at[0], ds.at[0])
        cp2.start()

        barrier = pltpu.get_barrier_semaphore()
        for o in range(1, N_DEV):
            pl.semaphore_signal(
                barrier,
                inc=1,
                device_id=(my_x, (yy + o) % N_DEV, my_z),
                device_id_type=pl.DeviceIdType.MESH,
            )
        pl.semaphore_wait(barrier, N_DEV - 1)

        rdmas = []
        for o in range(1, N_DEV):
            tgt = (my_x, (yy + o) % N_DEV, my_z)
            rd = pltpu.make_async_remote_copy(
                src_ref=sb_ref.at[o],
                dst_ref=recv_ref.at[N_DEV - o],
                send_sem=ds.at[o],
                recv_sem=dr.at[N_DEV - o],
                device_id=tgt,
                device_id_type=pl.DeviceIdType.MESH,
            )
            rd.start()
            rc = pltpu.make_async_remote_copy(
                src_ref=sc_ref,
                dst_ref=rc_ref.at[N_DEV - o],
                send_sem=cs.at[o],
                recv_sem=cr.at[N_DEV - o],
                device_id=tgt,
                device_id_type=pl.DeviceIdType.MESH,
            )
            rc.start()
            rdmas.append((rd, rc))
        cp.wait()
        cp2.wait()
        for rd, rc in rdmas:
            rd.wait()
            rc.wait()

    recvbuf, recvcnt = pl.pallas_call(
        body,
        out_shape=[
            jax.ShapeDtypeStruct((N_DEV, P, n), jnp.bfloat16),
            jax.ShapeDtypeStruct((N_DEV, 8, 128), jnp.int32),
        ],
        in_specs=[
            pl.BlockSpec(memory_space=pltpu.VMEM),
            pl.BlockSpec(memory_space=pltpu.VMEM),
        ],
        out_specs=[
            pl.BlockSpec(memory_space=pl.ANY),
            pl.BlockSpec(memory_space=pl.ANY),
        ],
        scratch_shapes=[
            pltpu.SemaphoreType.DMA((N_DEV,)),
            pltpu.SemaphoreType.DMA((N_DEV,)),
            pltpu.SemaphoreType.DMA((N_DEV,)),
            pltpu.SemaphoreType.DMA((N_DEV,)),
            pltpu.SemaphoreType.DMA,
        ],
        compiler_params=pltpu.CompilerParams(collective_id=0),
    )(sendbuf, sendcnt)

    srcs = jnp.arange(N_DEV)
    slot_of_src = (srcs - my_y) % N_DEV
    cnt_to_me = recvcnt[slot_of_src, 0, my_y]
    offs = jnp.concatenate(
        [jnp.zeros((1,), jnp.int32), jnp.cumsum(cnt_to_me).astype(jnp.int32)]
    )
    i = jnp.arange(m)
    j_of_i = jnp.searchsorted(offs[1:], i, side="right")
    flat = slot_of_src[j_of_i] * P + (i - offs[j_of_i])
    return recvbuf.reshape(N_DEV * P, n)[flat].astype(jnp.float32)


# device time: 142766 ns/iter; 4.7161x vs baseline; 1.4692x over previous
import jax
import jax.numpy as jnp
from jax import lax
from jax.experimental import pallas as pl
from jax.experimental.pallas import tpu as pltpu

N_DEV = 4
BITS = 13


def kernel(x, dest):
    m, n = x.shape
    xb = x.astype(jnp.bfloat16)
    dest = dest.astype(jnp.int32)
    counts = jnp.bincount(dest, length=N_DEV).astype(jnp.int32)
    order = jnp.argsort(dest, stable=True)
    xs = xb[order].reshape(m, 8, n // 8)
    cnt_in = jnp.zeros((1, 8), jnp.int32).at[0, :N_DEV].set(counts)

    def body(xs_ref, cnt_ref, out_ref, acv, acs, csend, crecv, c0, dsend, drecv, lsem):
        xx = lax.axis_index("x")
        yy = lax.axis_index("y")
        zz = lax.axis_index("z")

        cp = pltpu.make_async_copy(cnt_ref, acv.at[0], c0)
        cp.start()

        barrier = pltpu.get_barrier_semaphore()
        for o in range(1, N_DEV):
            pl.semaphore_signal(
                barrier,
                inc=1,
                device_id=(xx, (yy + o) % N_DEV, zz),
                device_id_type=pl.DeviceIdType.MESH,
            )
        pl.semaphore_wait(barrier, N_DEV - 1)

        cds = []
        for o in range(1, N_DEV):
            cd = pltpu.make_async_remote_copy(
                src_ref=cnt_ref,
                dst_ref=acv.at[N_DEV - o],
                send_sem=csend.at[o],
                recv_sem=crecv.at[N_DEV - o],
                device_id=(xx, (yy + o) % N_DEV, zz),
                device_id_type=pl.DeviceIdType.MESH,
            )
            cd.start()
            cds.append(cd)
        cp.wait()
        for cd in cds:
            cd.wait_recv()
        cp2 = pltpu.make_async_copy(acv, acs, c0)
        cp2.start()
        cp2.wait()
        for cd in cds:
            cd.wait_send()

        def cnt(j, d):
            return acs[(j - yy) % N_DEV, 0, d]

        my_c = [acs[0, 0, d] for d in range(N_DEV)]
        offs_local, base = [], []
        acc = jnp.int32(0)
        for d in range(N_DEV):
            offs_local.append(acc)
            acc = acc + my_c[d]
            b = jnp.int32(0)
            for j in range(N_DEV):
                b = b + jnp.where(jnp.int32(j) < yy, cnt(j, d), 0)
            base.append(b)

        def chunks(count, fn):
            off = jnp.int32(0)
            for b in reversed(range(BITS)):
                sz = 1 << b
                bit = (count & sz) != 0
                off_now = off

                @pl.when(bit)
                def _(off_now=off_now, sz=sz, fn=fn):
                    fn(off_now, sz)

                off = off + jnp.where(bit, jnp.int32(sz), 0)

        for d in range(N_DEV):

            @pl.when(d == yy)
            def _(d=d):
                def send_local(off, sz):
                    cpl = pltpu.make_async_copy(
                        xs_ref.at[pl.ds(offs_local[d] + off, sz)],
                        out_ref.at[pl.ds(base[d] + off, sz)],
                        lsem,
                    )
                    cpl.start()

                chunks(my_c[d], send_local)

            @pl.when(d != yy)
            def _(d=d):
                def send_remote(off, sz):
                    rd = pltpu.make_async_remote_copy(
                        src_ref=xs_ref.at[pl.ds(offs_local[d] + off, sz)],
                        dst_ref=out_ref.at[pl.ds(base[d] + off, sz)],
                        send_sem=dsend,
                        recv_sem=drecv,
                        device_id=(xx, d, zz),
                        device_id_type=pl.DeviceIdType.MESH,
                    )
                    rd.start()

                chunks(my_c[d], send_remote)

        for d in range(N_DEV):

            @pl.when(d == yy)
            def _(d=d):
                def wait_local(off, sz):
                    pltpu.make_async_copy(
                        xs_ref.at[pl.ds(0, sz)], out_ref.at[pl.ds(0, sz)], lsem
                    ).wait()

                chunks(my_c[d], wait_local)

        for d in range(N_DEV):

            @pl.when(d != yy)
            def _(d=d):
                def wait_send(off, sz):
                    pltpu.make_async_remote_copy(
                        src_ref=xs_ref.at[pl.ds(0, sz)],
                        dst_ref=out_ref.at[pl.ds(0, sz)],
                        send_sem=dsend,
                        recv_sem=drecv,
                        device_id=(xx, d, zz),
                        device_id_type=pl.DeviceIdType.MESH,
                    ).wait_send()

                chunks(my_c[d], wait_send)

        for j in range(N_DEV):

            @pl.when(j != yy)
            def _(j=j):
                def wait_recv(off, sz):
                    pltpu.make_async_remote_copy(
                        src_ref=xs_ref.at[pl.ds(0, sz)],
                        dst_ref=out_ref.at[pl.ds(0, sz)],
                        send_sem=dsend,
                        recv_sem=drecv,
                        device_id=(xx, j, zz),
                        device_id_type=pl.DeviceIdType.MESH,
                    ).wait_recv()

                chunks(cnt(j, yy), wait_recv)

    out = pl.pallas_call(
        body,
        out_shape=jax.ShapeDtypeStruct((m, 8, n // 8), jnp.bfloat16),
        in_specs=[
            pl.BlockSpec(memory_space=pltpu.VMEM),
            pl.BlockSpec(memory_space=pltpu.VMEM),
        ],
        out_specs=pl.BlockSpec(memory_space=pltpu.VMEM),
        scratch_shapes=[
            pltpu.VMEM((N_DEV, 1, 8), jnp.int32),
            pltpu.SMEM((N_DEV, 1, 8), jnp.int32),
            pltpu.SemaphoreType.DMA((N_DEV,)),
            pltpu.SemaphoreType.DMA((N_DEV,)),
            pltpu.SemaphoreType.DMA,
            pltpu.SemaphoreType.DMA,
            pltpu.SemaphoreType.DMA,
            pltpu.SemaphoreType.DMA,
        ],
        compiler_params=pltpu.CompilerParams(collective_id=0),
    )(xs, cnt_in)
    return out.reshape(m, n)
